# Initial kernel scaffold; baseline (speedup 1.0000x reference)
import jax
import jax.numpy as jnp
from jax.experimental import pallas as pl


def kernel(x, edge_index, en_w1, en_b1, en_w2, en_b2, ee_w1, ee_b1, ee_w2, ee_b2, un_w1, un_b1, un_w2, un_b2, ue_w1, ue_b1, ue_w2, ue_b2, ro_w1, ro_b1, ro_w2, ro_b2):
    raise NotImplementedError("write your pallas kernel here")


# speedup vs baseline: 3.1009x; 3.1009x over previous
import functools

import jax
import jax.numpy as jnp
from jax import lax
from jax.experimental import pallas as pl
from jax.experimental.pallas import tpu as pltpu
from jax.experimental.pallas import tpu_sc as plsc

N = 10000
E = 320000
F = 128
H = 32
L = 2

NC = 2
NS = 16
NW = NC * NS
CHUNK = 128
UNIT = 4
UE = CHUNK * UNIT
NUNITS = E // UE
NCHUNKS = E // CHUNK
ROWS_PER_TILE = N // NS

@functools.lru_cache(maxsize=1)
def _sc_kernels():
    mesh = plsc.VectorSubcoreMesh(
        core_axis_name="c", subcore_axis_name="s",
        num_cores=NC, num_subcores=NS)

    @functools.partial(
        pl.kernel,
        out_type=(
            jax.ShapeDtypeStruct((E, H), jnp.float32),
            jax.ShapeDtypeStruct((E, H), jnp.float32),
        ),
        mesh=mesh,
        compiler_params=pltpu.CompilerParams(use_tc_tiling_on_sc=False),
        scratch_types=[
            pltpu.VMEM((UNIT, CHUNK), jnp.int32),
            pltpu.VMEM((UNIT, CHUNK), jnp.int32),
            pltpu.VMEM((UE, H), jnp.float32),
            pltpu.VMEM((UE, H), jnp.float32),
            pltpu.SemaphoreType.DMA,
        ],
    )
    def gather2(t1, t2, src2d, dst2d, g1, g2, idx_s, idx_d, rows1, rows2,
                sem):
        c = lax.axis_index("c")
        s = lax.axis_index("s")
        w = s * NC + c
        lo = (NUNITS * w) // NW
        hi = (NUNITS * (w + 1)) // NW

        def body(u, carry):
            c0 = u * UNIT
            pltpu.sync_copy(src2d.at[pl.ds(c0, UNIT)], idx_s)
            pltpu.sync_copy(dst2d.at[pl.ds(c0, UNIT)], idx_d)
            for j in range(UNIT):
                pltpu.async_copy(
                    t1.at[idx_s.at[j]], rows1.at[pl.ds(j * CHUNK, CHUNK)],
                    sem).wait()
                pltpu.async_copy(
                    t2.at[idx_d.at[j]], rows2.at[pl.ds(j * CHUNK, CHUNK)],
                    sem).wait()
            pltpu.sync_copy(rows1, g1.at[pl.ds(u * UE, UE)])
            pltpu.sync_copy(rows2, g2.at[pl.ds(u * UE, UE)])
            return carry

        lax.fori_loop(lo, hi, body, 0)

    @functools.partial(
        pl.kernel,
        out_type=(
            jax.ShapeDtypeStruct((NC, N, H), jnp.float32),
            jax.ShapeDtypeStruct((NC, N, H), jnp.float32),
        ),
        mesh=mesh,
        compiler_params=pltpu.CompilerParams(use_tc_tiling_on_sc=False),
        scratch_types=[
            pltpu.VMEM((UNIT, CHUNK), jnp.int32),
            pltpu.VMEM((UNIT, CHUNK), jnp.int32),
            pltpu.VMEM((UE, H), jnp.float32),
            pltpu.VMEM_SHARED((N, H), jnp.float32),
            pltpu.VMEM_SHARED((N, H), jnp.float32),
            pltpu.SemaphoreType.DMA,
        ],
    )
    def scatter2(e, src2d, dst2d, zeros, inc_out, outg_out,
                 idx_s, idx_d, erows, acc_inc, acc_outg, sem):
        c = lax.axis_index("c")
        s = lax.axis_index("s")
        w = s * NC + c
        r0 = s * ROWS_PER_TILE
        pltpu.sync_copy(zeros.at[pl.ds(r0, ROWS_PER_TILE)],
                        acc_inc.at[pl.ds(r0, ROWS_PER_TILE)])
        pltpu.sync_copy(zeros.at[pl.ds(r0, ROWS_PER_TILE)],
                        acc_outg.at[pl.ds(r0, ROWS_PER_TILE)])
        plsc.subcore_barrier()

        lo = (NUNITS * w) // NW
        hi = (NUNITS * (w + 1)) // NW

        def body(u, carry):
            c0 = u * UNIT
            pltpu.sync_copy(src2d.at[pl.ds(c0, UNIT)], idx_s)
            pltpu.sync_copy(dst2d.at[pl.ds(c0, UNIT)], idx_d)
            pltpu.sync_copy(e.at[pl.ds(u * UE, UE)], erows)
            for j in range(UNIT):
                sl = erows.at[pl.ds(j * CHUNK, CHUNK)]
                pltpu.sync_copy(sl, acc_inc.at[idx_d.at[j]], add=True)
                pltpu.sync_copy(sl, acc_outg.at[idx_s.at[j]], add=True)
            return carry

        lax.fori_loop(lo, hi, body, 0)
        plsc.subcore_barrier()
        pltpu.sync_copy(acc_inc.at[pl.ds(r0, ROWS_PER_TILE)],
                        inc_out.at[c, pl.ds(r0, ROWS_PER_TILE)])
        pltpu.sync_copy(acc_outg.at[pl.ds(r0, ROWS_PER_TILE)],
                        outg_out.at[c, pl.ds(r0, ROWS_PER_TILE)])

    return gather2, scatter2


BN = 2000
BE = 8000


def _dot(a, b):
    return jnp.dot(a, b, preferred_element_type=jnp.float32)


def _node_prep_body(x_ref, w1, b1, w2, b2, wa, wb, h_ref, as_ref, ad_ref):
    t = jnp.maximum(_dot(x_ref[...], w1[...]) + b1[...], 0.0)
    h = _dot(t, w2[...]) + b2[...]
    h_ref[...] = h
    as_ref[...] = _dot(h, wa[...])
    ad_ref[...] = _dot(h, wb[...])


def _embed_body(g1, g2, b1, w2, b2, e_ref):
    z = jnp.maximum(g1[...] + g2[...] + b1[...], 0.0)
    e_ref[...] = _dot(z, w2[...]) + b2[...]


def _node_update_body(pinc, poutg, h_ref, wa, wb, wc, b1, w2, b2,
                      ua, ub, hn_ref, us_ref, ud_ref):
    inc = pinc[0] + pinc[1]
    outg = poutg[0] + poutg[1]
    h = h_ref[...]
    z = jnp.maximum(_dot(inc, wa[...]) + _dot(outg, wb[...]) +
                    _dot(h, wc[...]) + b1[...], 0.0)
    hn = h + _dot(z, w2[...]) + b2[...]
    hn_ref[...] = hn
    us_ref[...] = _dot(hn, ua[...])
    ud_ref[...] = _dot(hn, ub[...])


def _edge_update_body(e, g1, g2, wc, b1, w2, b2, eo_ref):
    z = jnp.maximum(_dot(e[...], wc[...]) + g1[...] + g2[...] + b1[...], 0.0)
    eo_ref[...] = e[...] + _dot(z, w2[...]) + b2[...]


def _edge_final_body(e, g1, g2, wc, b1, w2, b2, rw1, rb1, rw2, rb2, f_ref):
    z = jnp.maximum(_dot(e[...], wc[...]) + g1[...] + g2[...] + b1[...], 0.0)
    en = e[...] + _dot(z, w2[...]) + b2[...]
    r = jnp.maximum(_dot(en, rw1[...]) + rb1[...], 0.0)
    f_ref[...] = _dot(r, rw2[...]) + rb2[...]


def _full(shape):
    return pl.BlockSpec(shape, lambda i: tuple(0 for _ in shape))


def _nspec(width):
    return pl.BlockSpec((BN, width), lambda i: (i, 0))


def _espec(width):
    return pl.BlockSpec((BE, width), lambda i: (i, 0))


def _pspec():
    return pl.BlockSpec((NC, BN, H), lambda i: (0, i, 0))


def kernel(x, edge_index, en_w1, en_b1, en_w2, en_b2, ee_w1, ee_b1, ee_w2,
           ee_b2, un_w1, un_b1, un_w2, un_b2, ue_w1, ue_b1, ue_w2, ue_b2,
           ro_w1, ro_b1, ro_w2, ro_b2):
    _gather2, _scatter2 = _sc_kernels()
    src2d = edge_index[0].reshape(NCHUNKS, CHUNK)
    dst2d = edge_index[1].reshape(NCHUNKS, CHUNK)
    zeros = jnp.zeros((N, H), jnp.float32)

    ee_w1a, ee_w1b = ee_w1[:H], ee_w1[H:]
    un_w1a, un_w1b, un_w1c = un_w1[:H], un_w1[H:2 * H], un_w1[2 * H:]
    ue_w1a, ue_w1b, ue_w1c = ue_w1[:H], ue_w1[H:2 * H], ue_w1[2 * H:]
    en_b1r = en_b1.reshape(1, H)
    en_b2r = en_b2.reshape(1, H)
    ee_b1r = ee_b1.reshape(1, H)
    ee_b2r = ee_b2.reshape(1, H)
    un_b1r = un_b1.reshape(1, H)
    un_b2r = un_b2.reshape(1, H)
    ue_b1r = ue_b1.reshape(1, H)
    ue_b2r = ue_b2.reshape(1, H)
    ro_b1r = ro_b1.reshape(1, H)
    ro_b2r = ro_b2.reshape(1, 1)

    h, a_src, a_dst = pl.pallas_call(
        _node_prep_body,
        grid=(N // BN,),
        in_specs=[_nspec(F), _full((F, H)), _full((1, H)), _full((H, H)),
                  _full((1, H)), _full((H, H)), _full((H, H))],
        out_specs=[_nspec(H), _nspec(H), _nspec(H)],
        out_shape=[jax.ShapeDtypeStruct((N, H), jnp.float32)] * 3,
    )(x, en_w1, en_b1r, en_w2, en_b2r, ee_w1a, ee_w1b)

    g1, g2 = _gather2(a_src, a_dst, src2d, dst2d)

    e = pl.pallas_call(
        _embed_body,
        grid=(E // BE,),
        in_specs=[_espec(H), _espec(H), _full((1, H)), _full((H, H)),
                  _full((1, H))],
        out_specs=_espec(H),
        out_shape=jax.ShapeDtypeStruct((E, H), jnp.float32),
    )(g1, g2, ee_b1r, ee_w2, ee_b2r)

    flows = None
    for it in range(L):
        pinc, poutg = _scatter2(e, src2d, dst2d, zeros)
        h, u_src, u_dst = pl.pallas_call(
            _node_update_body,
            grid=(N // BN,),
            in_specs=[_pspec(), _pspec(), _nspec(H), _full((H, H)),
                      _full((H, H)), _full((H, H)), _full((1, H)),
                      _full((H, H)), _full((1, H)), _full((H, H)),
                      _full((H, H))],
            out_specs=[_nspec(H), _nspec(H), _nspec(H)],
            out_shape=[jax.ShapeDtypeStruct((N, H), jnp.float32)] * 3,
        )(pinc, poutg, h, un_w1a, un_w1b, un_w1c, un_b1r, un_w2, un_b2r,
          ue_w1a, ue_w1b)
        g1, g2 = _gather2(u_src, u_dst, src2d, dst2d)
        if it < L - 1:
            e = pl.pallas_call(
                _edge_update_body,
                grid=(E // BE,),
                in_specs=[_espec(H), _espec(H), _espec(H), _full((H, H)),
                          _full((1, H)), _full((H, H)), _full((1, H))],
                out_specs=_espec(H),
                out_shape=jax.ShapeDtypeStruct((E, H), jnp.float32),
            )(e, g1, g2, ue_w1c, ue_b1r, ue_w2, ue_b2r)
        else:
            flows = pl.pallas_call(
                _edge_final_body,
                grid=(E // BE,),
                in_specs=[_espec(H), _espec(H), _espec(H), _full((H, H)),
                          _full((1, H)), _full((H, H)), _full((1, H)),
                          _full((H, H)), _full((1, H)), _full((H, 1)),
                          _full((1, 1))],
                out_specs=_espec(1),
                out_shape=jax.ShapeDtypeStruct((E, 1), jnp.float32),
            )(e, g1, g2, ue_w1c, ue_b1r, ue_w2, ue_b2r,
              ro_w1, ro_b1r, ro_w2, ro_b2r)
    return flows
